# baseline (speedup 1.0000x reference)
import jax
import jax.numpy as jnp
from jax import lax
from jax.experimental import pallas as pl
from jax.experimental.pallas import tpu as pltpu

FEAT_ = 128
NE_ = 64
TOPK_ = 8
ROWS_ = 2048


def _relu(v):
    return jnp.maximum(v, 0.0)


def _dot(a, b):
    return lax.dot_general(a, b, (((1,), (0,)), ((), ())),
                           preferred_element_type=jnp.float32)


def _tree_reduce(v, op):
    while v.shape[0] > 8:
        h = v.shape[0] // 2
        v = op(v[:h, :], v[h:, :])
    for sh in (4, 2, 1):
        v = op(v, jnp.roll(v, -sh, axis=0))
    return v[0:1, :]


def _body(tT_ref, xbT_ref, w015T_ref, cwbT_ref, cbT_ref, fc4wbT_ref,
          hcatbT_ref, hbT_ref, outT_ref, idxT_ref):
    bf16 = jnp.bfloat16
    tT = tT_ref[...]
    xbT = xbT_ref[...]
    w015T = w015T_ref[...]
    cbT = cbT_ref[...]

    s0T = _relu(w015T[:, 0:1] * tT[0:1, :] + w015T[:, 3:4])
    s1T = _relu(w015T[:, 1:2] * tT[1:2, :] + w015T[:, 4:5])
    s5T = _relu(w015T[:, 2:3] * tT[2:3, :] + w015T[:, 5:6])

    cwbT = cwbT_ref[...]
    s2T = _relu(_dot(cwbT[:, 0:8], xbT[0:8, :]) + cbT[:, 0:1])
    s3T = _relu(_dot(cwbT[:, 8:16], xbT[8:16, :]) + cbT[:, 1:2])
    s4T = _relu(_dot(cwbT[:, 16:24], xbT[16:24, :]) + cbT[:, 2:3])

    mergeT = jnp.concatenate(
        [s0T.astype(bf16), s1T.astype(bf16), s2T.astype(bf16),
         s3T.astype(bf16), s4T.astype(bf16), s5T.astype(bf16)], axis=0)

    mhT = _dot(fc4wbT_ref[...], mergeT) + cbT[:, 3:4]
    lnT = _dot(hcatbT_ref[...], mhT.astype(bf16)) + hbT_ref[:, 0:1]

    logitsT = lnT[:NE_, :]
    nlT = lnT[NE_:2 * NE_, :]
    phaseT = lnT[2 * NE_:2 * NE_ + 1, :]
    softplusT = jnp.maximum(nlT, 0.0) + jnp.log1p(jnp.exp(-jnp.abs(nlT)))
    noisyT = logitsT + jnp.sin(phaseT) * softplusT

    iotaT = lax.broadcasted_iota(jnp.int32, noisyT.shape, 0)
    neg_inf = jnp.float32(-jnp.inf)
    work = noisyT
    sel = jnp.full_like(noisyT, neg_inf)
    m0 = jnp.max(work, axis=0, keepdims=True)
    idx_rows = []
    for k in range(TOPK_):
        am = jnp.argmax(work, axis=0, keepdims=True)
        oh = iotaT == am
        sel = jnp.where(oh, work, sel)
        work = jnp.where(oh, neg_inf, work)
        idx_rows.append(am.astype(jnp.int32))

    e = jnp.exp(sel - m0)
    outT_ref[...] = e / jnp.sum(e, axis=0, keepdims=True)
    idxT_ref[...] = jnp.concatenate(idx_rows, axis=0)


def kernel(x, fc1_w, fc1_b, fc2_w, fc2_b, conv1_w, conv1_b, conv2_w, conv2_b,
           conv3_w, conv3_b, fc3_w, fc3_b, fc4_w, fc4_b, tk_w, tk_b, nz_w, nz_b,
           pseudo_proj):
    B = x.shape[0]
    bf16, f32 = jnp.bfloat16, jnp.float32
    xf = x.reshape(B, 48)
    tT = jnp.concatenate(
        [xf[:, 7:8], xf[:, 15:16], xf[:, 47:48],
         jnp.zeros((B, 5), f32)], axis=1).T
    xbT = xf[:, 16:40].astype(bf16).T

    w015T = jnp.concatenate(
        [fc1_w.T, fc2_w.T, fc3_w.T, fc1_b[:, None], fc2_b[:, None],
         fc3_b[:, None], jnp.zeros((FEAT_, 2), f32)], axis=1)

    cwbT = jnp.concatenate(
        [conv1_w.T, conv2_w.T, conv3_w.T, jnp.zeros((FEAT_, 2), f32)],
        axis=1).astype(bf16)
    cbT = jnp.concatenate(
        [conv1_b[:, None], conv2_b[:, None], conv3_b[:, None],
         fc4_b[:, None], jnp.zeros((FEAT_, 4), f32)], axis=1)

    fc4wbT = fc4_w.T.astype(bf16)

    hcatbT = jnp.concatenate(
        [tk_w.T, nz_w.T, pseudo_proj.T, jnp.zeros((NE_ - 1, FEAT_), f32)],
        axis=0).astype(bf16)
    hbT = jnp.concatenate(
        [tk_b, nz_b, jnp.zeros((NE_,), f32)])[:, None].repeat(8, axis=1)

    grid = (B // ROWS_,)
    full = lambda shape: pl.BlockSpec(shape, lambda i: (0, 0))
    outT, idxT = pl.pallas_call(
        _body,
        grid=grid,
        in_specs=[
            pl.BlockSpec((8, ROWS_), lambda i: (0, i)),
            pl.BlockSpec((24, ROWS_), lambda i: (0, i)),
            full((FEAT_, 8)),
            full((FEAT_, 24)),
            full((FEAT_, 8)),
            full((FEAT_, 768)),
            full((192, FEAT_)),
            full((192, 8)),
        ],
        out_specs=[
            pl.BlockSpec((NE_, ROWS_), lambda i: (0, i)),
            pl.BlockSpec((TOPK_, ROWS_), lambda i: (0, i)),
        ],
        out_shape=[
            jax.ShapeDtypeStruct((NE_, B), jnp.float32),
            jax.ShapeDtypeStruct((TOPK_, B), jnp.int32),
        ],
        compiler_params=pltpu.CompilerParams(
            dimension_semantics=("arbitrary",)),
    )(tT, xbT, w015T, cwbT, cbT, fc4wbT, hcatbT, hbT)
    return (outT.T, idxT.T)


# speedup vs baseline: 1.1082x; 1.1082x over previous
import jax
import jax.numpy as jnp
from jax import lax
from jax.experimental import pallas as pl
from jax.experimental.pallas import tpu as pltpu

FEAT_ = 128
NE_ = 64
TOPK_ = 8
ROWS_ = 2048


def _relu(v):
    return jnp.maximum(v, 0.0)


def _dot(a, b):
    return lax.dot_general(a, b, (((1,), (0,)), ((), ())),
                           preferred_element_type=jnp.float32)


def _tree_reduce(v, op):
    while v.shape[0] > 8:
        h = v.shape[0] // 2
        v = op(v[:h, :], v[h:, :])
    for sh in (4, 2, 1):
        v = op(v, jnp.roll(v, -sh, axis=0))
    return v[0:1, :]


def _body(xfT_ref, w015T_ref, cwbT_ref, cbT_ref, fc4wbT_ref, hcatbT_ref,
          hbT_ref, outT_ref, idxT_ref):
    bf16 = jnp.bfloat16
    xfT = xfT_ref[...]
    w015T = w015T_ref[...]
    cbT = cbT_ref[...]

    s0T = _relu(w015T[:, 0:1] * xfT[7:8, :] + w015T[:, 3:4])
    s1T = _relu(w015T[:, 1:2] * xfT[15:16, :] + w015T[:, 4:5])
    s5T = _relu(w015T[:, 2:3] * xfT[47:48, :] + w015T[:, 5:6])

    xbT = xfT.astype(bf16)
    cwbT = cwbT_ref[...]
    s2T = _relu(_dot(cwbT[:, 0:8], xbT[16:24, :]) + cbT[:, 0:1])
    s3T = _relu(_dot(cwbT[:, 8:16], xbT[24:32, :]) + cbT[:, 1:2])
    s4T = _relu(_dot(cwbT[:, 16:24], xbT[32:40, :]) + cbT[:, 2:3])

    mergeT = jnp.concatenate(
        [s0T.astype(bf16), s1T.astype(bf16), s2T.astype(bf16),
         s3T.astype(bf16), s4T.astype(bf16), s5T.astype(bf16)], axis=0)

    mhT = _dot(fc4wbT_ref[...], mergeT) + cbT[:, 3:4]
    lnT = _dot(hcatbT_ref[...], mhT.astype(bf16)) + hbT_ref[:, 0:1]

    logitsT = lnT[:NE_, :]
    nlT = lnT[NE_:2 * NE_, :]
    phaseT = lnT[2 * NE_:2 * NE_ + 1, :]
    softplusT = jnp.maximum(nlT, 0.0) + jnp.log1p(jnp.exp(-jnp.abs(nlT)))
    noisyT = logitsT + jnp.sin(phaseT) * softplusT

    iotaT = lax.broadcasted_iota(jnp.int32, noisyT.shape, 0)
    neg_inf = jnp.float32(-jnp.inf)
    work = noisyT
    sel = jnp.full_like(noisyT, neg_inf)
    m0 = jnp.max(work, axis=0, keepdims=True)
    idx_rows = []
    for k in range(TOPK_):
        am = jnp.argmax(work, axis=0, keepdims=True)
        oh = iotaT == am
        sel = jnp.where(oh, work, sel)
        work = jnp.where(oh, neg_inf, work)
        idx_rows.append(am.astype(jnp.int32))

    e = jnp.exp(sel - m0)
    outT_ref[...] = e / jnp.sum(e, axis=0, keepdims=True)
    idxT_ref[...] = jnp.concatenate(idx_rows, axis=0)


def kernel(x, fc1_w, fc1_b, fc2_w, fc2_b, conv1_w, conv1_b, conv2_w, conv2_b,
           conv3_w, conv3_b, fc3_w, fc3_b, fc4_w, fc4_b, tk_w, tk_b, nz_w, nz_b,
           pseudo_proj):
    B = x.shape[0]
    bf16, f32 = jnp.bfloat16, jnp.float32
    xfT = x.reshape(B, 48).T

    w015T = jnp.concatenate(
        [fc1_w.T, fc2_w.T, fc3_w.T, fc1_b[:, None], fc2_b[:, None],
         fc3_b[:, None], jnp.zeros((FEAT_, 2), f32)], axis=1)

    cwbT = jnp.concatenate(
        [conv1_w.T, conv2_w.T, conv3_w.T, jnp.zeros((FEAT_, 2), f32)],
        axis=1).astype(bf16)
    cbT = jnp.concatenate(
        [conv1_b[:, None], conv2_b[:, None], conv3_b[:, None],
         fc4_b[:, None], jnp.zeros((FEAT_, 4), f32)], axis=1)

    fc4wbT = fc4_w.T.astype(bf16)

    hcatbT = jnp.concatenate(
        [tk_w.T, nz_w.T, pseudo_proj.T, jnp.zeros((NE_ - 1, FEAT_), f32)],
        axis=0).astype(bf16)
    hbT = jnp.concatenate(
        [tk_b, nz_b, jnp.zeros((NE_,), f32)])[:, None].repeat(8, axis=1)

    grid = (B // ROWS_,)
    full = lambda shape: pl.BlockSpec(shape, lambda i: (0, 0))
    outT, idxT = pl.pallas_call(
        _body,
        grid=grid,
        in_specs=[
            pl.BlockSpec((48, ROWS_), lambda i: (0, i)),
            full((FEAT_, 8)),
            full((FEAT_, 24)),
            full((FEAT_, 8)),
            full((FEAT_, 768)),
            full((192, FEAT_)),
            full((192, 8)),
        ],
        out_specs=[
            pl.BlockSpec((NE_, ROWS_), lambda i: (0, i)),
            pl.BlockSpec((TOPK_, ROWS_), lambda i: (0, i)),
        ],
        out_shape=[
            jax.ShapeDtypeStruct((NE_, B), jnp.float32),
            jax.ShapeDtypeStruct((TOPK_, B), jnp.int32),
        ],
        compiler_params=pltpu.CompilerParams(
            dimension_semantics=("parallel",)),
    )(xfT, w015T, cwbT, cbT, fc4wbT, hcatbT, hbT)
    return (outT.T, idxT.T)
